# baseline (speedup 1.0000x reference)
import jax
import jax.numpy as jnp
from jax.experimental import pallas as pl

_N = 1_000_000
_C = 16
_B = 8192


def _tf_bits(lo):
    ks0 = jnp.uint32(0)
    ks1 = jnp.uint32(42)
    ks2 = jnp.uint32(0x1BD11BDA ^ 42)
    ks = (ks0, ks1, ks2)
    x0 = jnp.full(lo.shape, ks0, jnp.uint32)
    x1 = lo + ks1
    rots = ((13, 15, 26, 6), (17, 29, 16, 24))
    for i in range(5):
        for r in rots[i % 2]:
            x0 = x0 + x1
            x1 = (x1 << jnp.uint32(r)) | (x1 >> jnp.uint32(32 - r))
            x1 = x0 ^ x1
        x0 = x0 + ks[(i + 1) % 3]
        x1 = x1 + ks[(i + 2) % 3] + jnp.uint32(i + 1)
    return x0 ^ x1


def _argmax16(v, sub8):
    a, b = v[0:8, :], v[8:16, :]
    take = b > a
    val = jnp.where(take, b, a)
    idx = jnp.where(take, sub8 + jnp.int32(8), sub8)
    for h in (4, 2, 1):
        va, vb = val[0:h, :], val[h:2 * h, :]
        ia, ib = idx[0:h, :], idx[h:2 * h, :]
        take = vb > va
        val = jnp.where(take, vb, va)
        idx = jnp.where(take, ib, ia)
    return idx


def _body(x_ref, base_ref, o_ref):
    i = pl.program_id(0)
    base = base_ref[...]
    x1 = base + (jnp.uint32(16 * _B) * jnp.uint32(i) + jnp.uint32(42))
    bits = _tf_bits_from_x1(x1)
    fb = (bits >> jnp.uint32(9)) | jnp.uint32(0x3F800000)
    f = jax.lax.bitcast_convert_type(fb, jnp.float32) - jnp.float32(1.0)
    u = f + jnp.float32(1e-20)
    g = -jnp.log(-jnp.log(u))
    v = x_ref[...] + g
    sub8 = (base[0:8, :] & jnp.uint32(15)).astype(jnp.int32)
    o_ref[...] = _argmax16(v, sub8)


def _tf_bits_from_x1(x1):
    ks0 = jnp.uint32(0)
    ks1 = jnp.uint32(42)
    ks2 = jnp.uint32(0x1BD11BDA ^ 42)
    ks = (ks0, ks1, ks2)
    x0 = jnp.zeros_like(x1)
    rots = ((13, 15, 26, 6), (17, 29, 16, 24))
    for i in range(5):
        for r in rots[i % 2]:
            x0 = x0 + x1
            x1 = (x1 << jnp.uint32(r)) | (x1 >> jnp.uint32(32 - r))
            x1 = x0 ^ x1
        x0 = x0 + ks[(i + 1) % 3]
        x1 = x1 + ks[(i + 2) % 3] + jnp.uint32(i + 1)
    return x0 ^ x1


import functools
from jax import lax
from jax.experimental.pallas import tpu as pltpu, tpu_sc as plsc

_R = 512
_WPW = 1536
_NS = 32 * _WPW
_NT = _N - _NS


def _ln(x):
    b = lax.bitcast_convert_type(x, jnp.uint32)
    mb = (b & jnp.uint32(0x007FFFFF)) | jnp.uint32(0x3F800000)
    big = mb >= jnp.uint32(0x3FB504F3)
    e = (b >> jnp.uint32(23)).astype(jnp.int32) - jnp.int32(127)
    e = e + jnp.where(big, jnp.int32(1), jnp.int32(0))
    m = lax.bitcast_convert_type(mb, jnp.float32)
    m = jnp.where(big, jnp.float32(0.5) * m, m)
    s = m - jnp.float32(1.0)
    z = s / (s + jnp.float32(2.0))
    w = z * z
    p = jnp.float32(1.0 / 9.0)
    p = p * w + jnp.float32(1.0 / 7.0)
    p = p * w + jnp.float32(1.0 / 5.0)
    p = p * w + jnp.float32(1.0 / 3.0)
    p = p * w + jnp.float32(1.0)
    return e.astype(jnp.float32) * jnp.float32(0.6931471805599453) + jnp.float32(2.0) * z * p


def _sc_part(lgs):
    mesh = plsc.VectorSubcoreMesh(core_axis_name="c", subcore_axis_name="s")

    @functools.partial(
        pl.kernel, mesh=mesh,
        out_type=jax.ShapeDtypeStruct((_NS,), jnp.int32),
        scratch_types=[
            pltpu.VMEM((_R, _C), jnp.float32),
            pltpu.VMEM((_R,), jnp.int32),
        ],
    )
    def k(lg_hbm, out_hbm, buf, obuf):
        wid = lax.axis_index("s") * 2 + lax.axis_index("c")
        i16iota = lax.iota(jnp.int32, 16)
        u16iota = i16iota.astype(jnp.uint32)

        def chunk_body(ch, carry):
            base = wid * _WPW + ch * _R
            pltpu.sync_copy(lg_hbm.at[pl.ds(base, _R)], buf)

            def group_body(gi, c2):
                res = lax.broadcast(jnp.int32(0), (16,))
                for j in range(16):
                    r = gi * jnp.int32(16) + jnp.int32(j)
                    lo = (jnp.uint32(16) * (base + r + _NT).astype(jnp.uint32)) + u16iota
                    bits = _tf_bits_from_x1(lo + jnp.uint32(42))
                    fb = (bits >> jnp.uint32(9)) | jnp.uint32(0x3F800000)
                    f = lax.bitcast_convert_type(fb, jnp.float32) - jnp.float32(1.0)
                    u = f + jnp.float32(1e-20)
                    g = -_ln(-_ln(u))
                    v = buf[r] + g
                    vals = [v[c] for c in range(16)]
                    idxs = [jnp.int32(c) for c in range(16)]
                    while len(vals) > 1:
                        nv, ni = [], []
                        for a in range(0, len(vals), 2):
                            take = vals[a + 1] > vals[a]
                            nv.append(jnp.where(take, vals[a + 1], vals[a]))
                            ni.append(jnp.where(take, idxs[a + 1], idxs[a]))
                        vals, idxs = nv, ni
                    res = jnp.where(i16iota == jnp.int32(j), lax.broadcast(idxs[0], (16,)), res)
                obuf[pl.ds(gi * jnp.int32(16), 16)] = res
                return c2

            lax.fori_loop(0, _R // 16, group_body, jnp.int32(0))
            pltpu.sync_copy(obuf, out_hbm.at[pl.ds(base, _R)])
            return carry

        lax.fori_loop(0, _WPW // _R, chunk_body, jnp.int32(0))

    return k(lgs)


def kernel(logits):
    sc_out = _sc_part(logits[_NT:])
    lt = logits[:_NT].T
    lane = jax.lax.broadcasted_iota(jnp.uint32, (_C, _B), 1)
    sub = jax.lax.broadcasted_iota(jnp.uint32, (_C, _B), 0)
    base = lane * jnp.uint32(_C) + sub
    tc_out = pl.pallas_call(
        _body,
        grid=(pl.cdiv(_NT, _B),),
        in_specs=[
            pl.BlockSpec((_C, _B), lambda i: (0, i)),
            pl.BlockSpec((_C, _B), lambda i: (0, 0)),
        ],
        out_specs=pl.BlockSpec((1, _B), lambda i: (0, i)),
        out_shape=jax.ShapeDtypeStruct((1, _NT), jnp.int32),
    )(lt, base)
    return jnp.concatenate([tc_out.reshape(_NT), sc_out])


# speedup vs baseline: 1.1659x; 1.1659x over previous
import jax
import jax.numpy as jnp
from jax.experimental import pallas as pl

_N = 1_000_000
_C = 16
_B = 8192


def _tf_bits(lo):
    ks0 = jnp.uint32(0)
    ks1 = jnp.uint32(42)
    ks2 = jnp.uint32(0x1BD11BDA ^ 42)
    ks = (ks0, ks1, ks2)
    x0 = jnp.full(lo.shape, ks0, jnp.uint32)
    x1 = lo + ks1
    rots = ((13, 15, 26, 6), (17, 29, 16, 24))
    for i in range(5):
        for r in rots[i % 2]:
            x0 = x0 + x1
            x1 = (x1 << jnp.uint32(r)) | (x1 >> jnp.uint32(32 - r))
            x1 = x0 ^ x1
        x0 = x0 + ks[(i + 1) % 3]
        x1 = x1 + ks[(i + 2) % 3] + jnp.uint32(i + 1)
    return x0 ^ x1


def _argmax16(v, sub8):
    a, b = v[0:8, :], v[8:16, :]
    take = b > a
    val = jnp.where(take, b, a)
    idx = jnp.where(take, sub8 + jnp.int32(8), sub8)
    for h in (4, 2, 1):
        va, vb = val[0:h, :], val[h:2 * h, :]
        ia, ib = idx[0:h, :], idx[h:2 * h, :]
        take = vb > va
        val = jnp.where(take, vb, va)
        idx = jnp.where(take, ib, ia)
    return idx


def _body(x_ref, base_ref, o_ref):
    i = pl.program_id(0)
    base = base_ref[...]
    x1 = base + (jnp.uint32(16 * _B) * jnp.uint32(i) + jnp.uint32(42))
    bits = _tf_bits_from_x1(x1)
    fb = (bits >> jnp.uint32(9)) | jnp.uint32(0x3F800000)
    f = jax.lax.bitcast_convert_type(fb, jnp.float32) - jnp.float32(1.0)
    u = f + jnp.float32(1e-20)
    g = -jnp.log(-jnp.log(u))
    v = x_ref[...] + g
    sub8 = (base[0:8, :] & jnp.uint32(15)).astype(jnp.int32)
    o_ref[...] = _argmax16(v, sub8)


def _tf_bits_from_x1(x1):
    ks0 = jnp.uint32(0)
    ks1 = jnp.uint32(42)
    ks2 = jnp.uint32(0x1BD11BDA ^ 42)
    ks = (ks0, ks1, ks2)
    x0 = jnp.zeros_like(x1)
    rots = ((13, 15, 26, 6), (17, 29, 16, 24))
    for i in range(5):
        for r in rots[i % 2]:
            x0 = x0 + x1
            x1 = (x1 << jnp.uint32(r)) | (x1 >> jnp.uint32(32 - r))
            x1 = x0 ^ x1
        x0 = x0 + ks[(i + 1) % 3]
        x1 = x1 + ks[(i + 2) % 3] + jnp.uint32(i + 1)
    return x0 ^ x1


def kernel(logits):
    lt = logits.T
    lane = jax.lax.broadcasted_iota(jnp.uint32, (_C, _B), 1)
    sub = jax.lax.broadcasted_iota(jnp.uint32, (_C, _B), 0)
    base = lane * jnp.uint32(_C) + sub
    out = pl.pallas_call(
        _body,
        grid=(pl.cdiv(_N, _B),),
        in_specs=[
            pl.BlockSpec((_C, _B), lambda i: (0, i)),
            pl.BlockSpec((_C, _B), lambda i: (0, 0)),
        ],
        out_specs=pl.BlockSpec((1, _B), lambda i: (0, i)),
        out_shape=jax.ShapeDtypeStruct((1, _N), jnp.int32),
    )(lt, base)
    return out.reshape(_N)


# speedup vs baseline: 1.1663x; 1.0003x over previous
import jax
import jax.numpy as jnp
from jax.experimental import pallas as pl

_N = 1_000_000
_C = 16
_B = 8192


def _argmax16(v, sub8):
    a, b = v[0:8, :], v[8:16, :]
    take = b > a
    val = jnp.where(take, b, a)
    idx = jnp.where(take, sub8 + jnp.int32(8), sub8)
    for h in (4, 2, 1):
        va, vb = val[0:h, :], val[h:2 * h, :]
        ia, ib = idx[0:h, :], idx[h:2 * h, :]
        take = vb > va
        val = jnp.where(take, vb, va)
        idx = jnp.where(take, ib, ia)
    return idx


def _body(x_ref, base_ref, o_ref):
    i = pl.program_id(0)
    base = base_ref[...]
    x1 = base + (jnp.uint32(16 * _B) * jnp.uint32(i) + jnp.uint32(42))
    bits = _tf_bits_from_x1(x1)
    fb = (bits >> jnp.uint32(9)) | jnp.uint32(0x3F800000)
    f = jax.lax.bitcast_convert_type(fb, jnp.float32) - jnp.float32(1.0)
    u = f + jnp.float32(1e-20)
    g = -jnp.log(-jnp.log(u))
    v = x_ref[...] + g
    sub8 = (base[0:8, :] & jnp.uint32(15)).astype(jnp.int32)
    o_ref[...] = _argmax16(v, sub8)


def _tf_bits_from_x1(x1):
    ks0 = jnp.uint32(0)
    ks1 = jnp.uint32(42)
    ks2 = jnp.uint32(0x1BD11BDA ^ 42)
    ks = (ks0, ks1, ks2)
    x0 = jnp.zeros_like(x1)
    rots = ((13, 15, 26, 6), (17, 29, 16, 24))
    for i in range(5):
        for r in rots[i % 2]:
            x0 = x0 + x1
            x1 = (x1 << jnp.uint32(r)) | (x1 >> jnp.uint32(32 - r))
            x1 = x0 ^ x1
        x0 = x0 + ks[(i + 1) % 3]
        x1 = x1 + ks[(i + 2) % 3] + jnp.uint32(i + 1)
    return x0 ^ x1


def kernel(logits):
    lt = logits.T
    lane = jax.lax.broadcasted_iota(jnp.uint32, (_C, _B), 1)
    sub = jax.lax.broadcasted_iota(jnp.uint32, (_C, _B), 0)
    base = lane * jnp.uint32(_C) + sub
    out = pl.pallas_call(
        _body,
        grid=(pl.cdiv(_N, _B),),
        in_specs=[
            pl.BlockSpec((_C, _B), lambda i: (0, i)),
            pl.BlockSpec((_C, _B), lambda i: (0, 0)),
        ],
        out_specs=pl.BlockSpec((1, _B), lambda i: (0, i)),
        out_shape=jax.ShapeDtypeStruct((1, _N), jnp.int32),
    )(lt, base)
    return out.reshape(_N)
